# Initial kernel scaffold; baseline (speedup 1.0000x reference)
import jax
import jax.numpy as jnp
from jax.experimental import pallas as pl


def kernel(features, weight, bias, edge_index, offset_id):
    raise NotImplementedError("write your pallas kernel here")


# speedup vs baseline: 2.7850x; 2.7850x over previous
import functools

import jax
import jax.numpy as jnp
from jax import lax
from jax.experimental import pallas as pl
from jax.experimental.pallas import tpu as pltpu
from jax.experimental.pallas import tpu_sc as plsc


def _transform_stage(features, weight, nc):
    n, nin = features.shape
    fv, _, nout = weight.shape
    noutc = nout // nc
    wsplit = jnp.moveaxis(weight.reshape(fv, nin, nc, noutc), 2, 0)

    def body(x_ref, w_ref, o_ref):
        o_ref[0] = jnp.dot(
            x_ref[...], w_ref[0, 0], preferred_element_type=jnp.float32
        )

    return pl.pallas_call(
        body,
        grid=(fv, nc),
        in_specs=[
            pl.BlockSpec((n, nin), lambda f, c: (0, 0)),
            pl.BlockSpec((1, 1, nin, noutc), lambda f, c: (c, f, 0, 0)),
        ],
        out_specs=pl.BlockSpec((1, n, noutc), lambda f, c: (c, f, 0)),
        out_shape=jax.ShapeDtypeStruct((nc, fv * n, noutc), jnp.float32),
    )(features, wsplit)


def _combine_stage(partials, bias):
    nc, n, noutc = partials.shape

    def body(p_ref, b_ref, o_ref):
        o_ref[...] = (
            jnp.concatenate([p_ref[c] for c in range(nc)], axis=-1) + b_ref[...]
        )

    return pl.pallas_call(
        body,
        in_specs=[
            pl.BlockSpec((nc, n, noutc), lambda: (0, 0, 0)),
            pl.BlockSpec((1, nc * noutc), lambda: (0, 0)),
        ],
        out_specs=pl.BlockSpec((n, nc * noutc), lambda: (0, 0)),
        out_shape=jax.ShapeDtypeStruct((n, nc * noutc), jnp.float32),
    )(partials, bias.reshape(1, nc * noutc))


def _scatter_stage(transformed, gidx, dst, nacc, nc, ns, chunk, chunks_per_tile):
    noutc = transformed.shape[1]
    table_rows_per_core = transformed.shape[0] // nc
    rpt = nacc // ns
    lanes = noutc // 16
    mesh = plsc.VectorSubcoreMesh(core_axis_name="c", subcore_axis_name="s")

    @functools.partial(
        pl.kernel,
        mesh=mesh,
        out_type=jax.ShapeDtypeStruct((nc, nacc, noutc), jnp.float32),
        scratch_types=[
            pltpu.VMEM((chunk,), jnp.int32),
            pltpu.VMEM((chunk,), jnp.int32),
            pltpu.VMEM((chunk, noutc), jnp.float32),
            pltpu.VMEM((rpt, noutc), jnp.float32),
            pltpu.VMEM_SHARED((nacc, noutc), jnp.float32),
            pltpu.SemaphoreType.DMA,
        ],
        compiler_params=pltpu.CompilerParams(use_tc_tiling_on_sc=False),
    )
    def sc_fn(tr_hbm, gidx_hbm, dst_hbm, part_hbm, g0, d0, r0, z0, acc, sem0):
        cid = lax.axis_index("c")
        sid = lax.axis_index("s")
        tile_base = sid * (chunks_per_tile * chunk)
        coff = cid * table_rows_per_core

        zvec = jnp.zeros((16,), jnp.float32)

        def zbody(i, _):
            z0[i // lanes, pl.ds((i % lanes) * 16, 16)] = zvec
            return 0

        lax.fori_loop(0, rpt * lanes, zbody, 0)
        pltpu.sync_copy(z0, acc.at[pl.ds(sid * rpt, rpt)])
        plsc.subcore_barrier()

        def body(j, _):
            base = tile_base + j * chunk
            pltpu.sync_copy(gidx_hbm.at[pl.ds(base, chunk)], g0)
            pltpu.sync_copy(dst_hbm.at[pl.ds(base, chunk)], d0)
            for i in range(chunk // 16):
                sl = pl.ds(i * 16, 16)
                g0[sl] = g0[sl] + coff
            pltpu.async_copy(tr_hbm.at[g0], r0, sem0).wait()
            pltpu.sync_copy(r0, acc.at[d0], add=True)
            return 0

        lax.fori_loop(0, chunks_per_tile, body, 0)
        plsc.subcore_barrier()

        pltpu.sync_copy(
            acc.at[pl.ds(sid * rpt, rpt)],
            part_hbm.at[cid, pl.ds(sid * rpt, rpt)],
        )

    return sc_fn(transformed, gidx, dst)


def kernel(features, weight, bias, edge_index, offset_id):
    n, nin = features.shape
    fv, _, nout = weight.shape
    e = edge_index.shape[1]

    info = plsc.get_sparse_core_info()
    nc, ns = info.num_cores, info.num_subcores

    chunk = 128
    chunks_per_tile = -(-e // (chunk * ns))
    ep = chunks_per_tile * chunk * ns

    rpt = -(-(n + 1) // (ns * chunk)) * chunk
    nacc = rpt * ns

    src = edge_index[0].astype(jnp.int32)
    dst = edge_index[1].astype(jnp.int32)
    off = offset_id.astype(jnp.int32)
    gidx = off * n + src
    pad = ep - e
    gidx_p = jnp.concatenate([gidx, jnp.zeros((pad,), jnp.int32)])
    dst_p = jnp.concatenate([dst, jnp.full((pad,), n, jnp.int32)])

    transformed = _transform_stage(features, weight, nc)
    partials = _scatter_stage(
        transformed.reshape(nc * fv * n, nout // nc),
        gidx_p,
        dst_p,
        nacc,
        nc,
        ns,
        chunk,
        chunks_per_tile,
    )
    return _combine_stage(partials[:, :n], bias)


# speedup vs baseline: 3.3795x; 1.2135x over previous
import functools

import jax
import jax.numpy as jnp
from jax import lax
from jax.experimental import pallas as pl
from jax.experimental.pallas import tpu as pltpu
from jax.experimental.pallas import tpu_sc as plsc


def _transform_stage(features, weight, nc):
    n, nin = features.shape
    fv, _, nout = weight.shape
    noutc = nout // nc
    wsplit = jnp.moveaxis(weight.reshape(fv, nin, nc, noutc), 2, 0)

    def body(x_ref, w_ref, o_ref):
        o_ref[0] = jnp.dot(
            x_ref[...], w_ref[0, 0], preferred_element_type=jnp.float32
        )

    return pl.pallas_call(
        body,
        grid=(fv, nc),
        in_specs=[
            pl.BlockSpec((n, nin), lambda f, c: (0, 0)),
            pl.BlockSpec((1, 1, nin, noutc), lambda f, c: (c, f, 0, 0)),
        ],
        out_specs=pl.BlockSpec((1, n, noutc), lambda f, c: (c, f, 0)),
        out_shape=jax.ShapeDtypeStruct((nc, fv * n, noutc), jnp.float32),
    )(features, wsplit)


def _combine_stage(partials, bias):
    nc, n, noutc = partials.shape

    def body(p_ref, b_ref, o_ref):
        o_ref[...] = (
            jnp.concatenate([p_ref[c] for c in range(nc)], axis=-1) + b_ref[...]
        )

    return pl.pallas_call(
        body,
        in_specs=[
            pl.BlockSpec((nc, n, noutc), lambda: (0, 0, 0)),
            pl.BlockSpec((1, nc * noutc), lambda: (0, 0)),
        ],
        out_specs=pl.BlockSpec((n, nc * noutc), lambda: (0, 0)),
        out_shape=jax.ShapeDtypeStruct((n, nc * noutc), jnp.float32),
    )(partials, bias.reshape(1, nc * noutc))


def _scatter_stage(transformed, gidx, dst, nacc, nc, ns, chunk, cpt):
    noutc = transformed.shape[1]
    table_rows_per_core = transformed.shape[0] // nc
    rpt = nacc // ns
    lanes = noutc // 16
    mesh = plsc.VectorSubcoreMesh(core_axis_name="c", subcore_axis_name="s")

    @functools.partial(
        pl.kernel,
        mesh=mesh,
        out_type=jax.ShapeDtypeStruct((nc, nacc, noutc), jnp.float32),
        scratch_types=[
            pltpu.VMEM((cpt, chunk), jnp.int32),
            pltpu.VMEM((cpt, chunk), jnp.int32),
            pltpu.VMEM((chunk, noutc), jnp.float32),
            pltpu.VMEM((chunk, noutc), jnp.float32),
            pltpu.VMEM_SHARED((nacc, noutc), jnp.float32),
            pltpu.SemaphoreType.DMA,
            pltpu.SemaphoreType.DMA,
        ],
        compiler_params=pltpu.CompilerParams(use_tc_tiling_on_sc=False),
    )
    def sc_fn(tr_hbm, gidx_hbm, dst_hbm, part_hbm, gall, dall, r0, r1,
              acc, sem0, sem1):
        cid = lax.axis_index("c")
        sid = lax.axis_index("s")
        coff = cid * table_rows_per_core

        pltpu.sync_copy(gidx_hbm.at[pl.ds(sid * cpt, cpt)], gall)
        pltpu.sync_copy(dst_hbm.at[pl.ds(sid * cpt, cpt)], dall)

        def cbody(i, _):
            sl = pl.ds((i % (chunk // 16)) * 16, 16)
            gall[i // (chunk // 16), sl] = gall[i // (chunk // 16), sl] + coff
            return 0

        lax.fori_loop(0, cpt * (chunk // 16), cbody, 0)

        zvec = jnp.zeros((16,), jnp.float32)

        def zbody(i, _):
            r0[i // lanes, pl.ds((i % lanes) * 16, 16)] = zvec
            return 0

        lax.fori_loop(0, chunk * lanes, zbody, 0)
        for q in range(rpt // chunk):
            pltpu.sync_copy(r0, acc.at[pl.ds(sid * rpt + q * chunk, chunk)])
        plsc.subcore_barrier()

        def gather_start(j, rbuf, sem):
            pltpu.async_copy(tr_hbm.at[gall.at[j]], rbuf, sem)

        def gather_wait(j, rbuf, sem):
            pltpu.make_async_copy(tr_hbm.at[gall.at[j]], rbuf, sem).wait()

        gather_start(0, r0, sem0)

        def body(t, _):
            j0 = 2 * t
            j1 = 2 * t + 1
            j2 = 2 * t + 2
            gather_wait(j0, r0, sem0)
            gather_start(j1, r1, sem1)
            pltpu.sync_copy(r0, acc.at[dall.at[j0]], add=True)
            gather_wait(j1, r1, sem1)

            @pl.when(j2 < cpt)
            def _():
                gather_start(j2, r0, sem0)

            pltpu.sync_copy(r1, acc.at[dall.at[j1]], add=True)
            return 0

        lax.fori_loop(0, cpt // 2, body, 0)
        plsc.subcore_barrier()

        pltpu.sync_copy(
            acc.at[pl.ds(sid * rpt, rpt)],
            part_hbm.at[cid, pl.ds(sid * rpt, rpt)],
        )

    return sc_fn(transformed, gidx, dst)


def kernel(features, weight, bias, edge_index, offset_id):
    n, nin = features.shape
    fv, _, nout = weight.shape
    e = edge_index.shape[1]

    info = plsc.get_sparse_core_info()
    nc, ns = info.num_cores, info.num_subcores

    chunk = 128
    cpt = -(-e // (chunk * ns))
    cpt += cpt % 2
    ep = cpt * chunk * ns

    rpt = -(-(n + 1) // (ns * chunk)) * chunk
    nacc = rpt * ns

    src = edge_index[0].astype(jnp.int32)
    dst = edge_index[1].astype(jnp.int32)
    off = offset_id.astype(jnp.int32)
    gidx = off * n + src
    pad = ep - e
    gidx_p = jnp.concatenate([gidx, jnp.zeros((pad,), jnp.int32)])
    dst_p = jnp.concatenate([dst, jnp.full((pad,), n, jnp.int32)])

    transformed = _transform_stage(features, weight, nc)
    partials = _scatter_stage(
        transformed.reshape(nc * fv * n, nout // nc),
        gidx_p.reshape(ep // chunk, chunk),
        dst_p.reshape(ep // chunk, chunk),
        nacc,
        nc,
        ns,
        chunk,
        cpt,
    )
    return _combine_stage(partials[:, :n], bias)
